# Initial kernel scaffold; baseline (speedup 1.0000x reference)
import jax
import jax.numpy as jnp
from jax.experimental import pallas as pl


def kernel(y_true, y_pred):
    raise NotImplementedError("write your pallas kernel here")


# speedup vs baseline: 8.5068x; 8.5068x over previous
import functools

import jax
import jax.numpy as jnp
from jax import lax
from jax.experimental import pallas as pl
from jax.experimental.pallas import tpu as pltpu
from jax.experimental.pallas import tpu_sc as plsc

B = 8
C = 3
H = 512
W = 512
NPIX = H * W
NCHUNK = 4
ROWS = H // NCHUNK
K_RANK = int(W * H * 0.4 - 1)
N_TOTAL = B * C * NPIX
SIGN = -2**31

NB1 = 2048
NB2 = 2048
NB3 = 1024


def _prep_body(yt_ref, yp_ref, key_ref, a_ref):
    yp = yp_ref[0]
    yt = yt_ref[0]
    g = (yp[0] + yp[1] + yp[2]) / 3.0
    bits = lax.bitcast_convert_type(g, jnp.int32)
    ukey = jnp.where(bits < 0, ~bits, bits ^ SIGN)
    key_ref[0, 0] = ukey
    a_ref[0, 0] = (jnp.abs(yp[0] - yt[0]) + jnp.abs(yp[1] - yt[1])
                   + jnp.abs(yp[2] - yt[2]))


def _prep(y_true, y_pred):
    return pl.pallas_call(
        _prep_body,
        grid=(B, NCHUNK),
        in_specs=[
            pl.BlockSpec((1, C, ROWS, W), lambda b, q: (b, 0, q, 0)),
            pl.BlockSpec((1, C, ROWS, W), lambda b, q: (b, 0, q, 0)),
        ],
        out_specs=[
            pl.BlockSpec((1, 1, ROWS, W), lambda b, q: (b, q, 0, 0)),
            pl.BlockSpec((1, 1, ROWS, W), lambda b, q: (b, q, 0, 0)),
        ],
        out_shape=[
            jax.ShapeDtypeStruct((B, NCHUNK, ROWS, W), jnp.int32),
            jax.ShapeDtypeStruct((B, NCHUNK, ROWS, W), jnp.float32),
        ],
    )(y_true, y_pred)


def _select_body(keys_hbm, t_hbm, keys_v, hist_v, red_v, tmp_v, tout_v, board):
    c = lax.axis_index("c")
    s = lax.axis_index("s")
    img_in_core = lax.shift_right_logical(s, 2)
    q = lax.bitwise_and(s, 3)
    b = c * 4 + img_in_core

    pltpu.sync_copy(keys_hbm.at[b, q], keys_v)

    lane = lax.iota(jnp.int32, 16)
    ones = jnp.ones((16,), jnp.int32)
    zeros16 = jnp.zeros((16,), jnp.int32)

    def zero_hist(nbins):
        def zr(i, _):
            hist_v[pl.ds(i * 16, 16)] = zeros16
            return 0
        lax.fori_loop(0, nbins * 16 // 16, zr, 0)

    def scan_level(nbins, shift, dmask, prefix_shift, prefix_val):
        lp = lane * nbins

        def body(r, _):
            for cc in range(W // 16):
                v = keys_v[r, pl.ds(cc * 16, 16)]
                d = lax.shift_right_logical(v, shift)
                if dmask is not None:
                    d = lax.bitwise_and(d, dmask)
                idx = d + lp
                if prefix_val is None:
                    plsc.addupdate_scatter(hist_v, [idx], ones)
                else:
                    pm = lax.shift_right_logical(v, prefix_shift) == prefix_val
                    plsc.addupdate_scatter(hist_v, [idx], ones, mask=pm)
            return 0

        lax.fori_loop(0, ROWS, body, 0)

    def reduce_and_combine(nbins):
        def red(j, _):
            acc = hist_v[pl.ds(j * 16, 16)]
            for p in range(1, 16):
                acc = acc + hist_v[pl.ds(p * nbins + j * 16, 16)]
            red_v[pl.ds(j * 16, 16)] = acc
            return 0
        lax.fori_loop(0, nbins // 16, red, 0)
        plsc.subcore_barrier()
        pltpu.sync_copy(red_v.at[pl.ds(0, nbins)], board.at[s, pl.ds(0, nbins)])
        plsc.subcore_barrier()
        base = img_in_core * 4
        pltpu.sync_copy(board.at[base, pl.ds(0, nbins)], red_v.at[pl.ds(0, nbins)])
        for qq in range(1, 4):
            pltpu.sync_copy(board.at[base + qq, pl.ds(0, nbins)],
                            tmp_v.at[pl.ds(0, nbins)])

            def addup(j, _):
                red_v[pl.ds(j * 16, 16)] = (red_v[pl.ds(j * 16, 16)]
                                            + tmp_v[pl.ds(j * 16, 16)])
                return 0
            lax.fori_loop(0, nbins // 16, addup, 0)

    def find_bin(nbins, k):
        def fb(j, carry):
            cnt, pb, run = carry
            v = red_v[pl.ds(j * 16, 16)]
            incl = plsc.cumsum(v) + run
            le = incl <= k
            cnt = cnt + jnp.sum(jnp.where(le, 1, 0))
            pb = jnp.maximum(pb, jnp.max(jnp.where(le, incl, 0)))
            run = jnp.max(incl)
            return (cnt, pb, run)

        z = jnp.int32(0)
        cnt, pb, _ = lax.fori_loop(0, nbins // 16, fb, (z, z, z))
        return cnt, k - pb

    zero_hist(NB1)
    scan_level(NB1, 21, None, None, None)
    reduce_and_combine(NB1)
    d1, r1 = find_bin(NB1, jnp.int32(K_RANK))

    zero_hist(NB2)
    d1v = jnp.full((16,), d1, jnp.int32)
    scan_level(NB2, 10, jnp.int32(NB2 - 1), 21, d1v)
    reduce_and_combine(NB2)
    d2, r2 = find_bin(NB2, r1)

    zero_hist(NB3)
    d12 = d1 * NB2 + d2
    d12v = jnp.full((16,), d12, jnp.int32)
    scan_level(NB3, 0, jnp.int32(NB3 - 1), 10, d12v)
    reduce_and_combine(NB3)
    d3, _ = find_bin(NB3, r2)

    t_u = lax.shift_left(d1, 21) + lax.shift_left(d2, 10) + d3
    tout_v[...] = jnp.full((16,), t_u, jnp.int32)

    @pl.when(q == 0)
    def _():
        pltpu.sync_copy(tout_v, t_hbm.at[b])


@functools.cache
def _select():
    mesh = plsc.VectorSubcoreMesh(core_axis_name="c", subcore_axis_name="s")
    return pl.kernel(
        _select_body,
        out_type=jax.ShapeDtypeStruct((B, 16), jnp.int32),
        mesh=mesh,
        scratch_types=[
            pltpu.VMEM((ROWS, W), jnp.int32),
            pltpu.VMEM((16 * NB1,), jnp.int32),
            pltpu.VMEM((NB1,), jnp.int32),
            pltpu.VMEM((NB1,), jnp.int32),
            pltpu.VMEM((16,), jnp.int32),
            pltpu.VMEM_SHARED((16, NB1), jnp.int32),
        ],
        compiler_params=pltpu.CompilerParams(needs_layout_passes=False),
    )


def _loss_body(ts_ref, key_ref, a_ref, out_ref):
    b = pl.program_id(0)
    q = pl.program_id(1)

    @pl.when(jnp.logical_and(b == 0, q == 0))
    def _():
        out_ref[...] = jnp.zeros_like(out_ref)

    t_s = ts_ref[b, 0] ^ SIGN
    skey = key_ref[0, 0] ^ SIGN
    wgt = jnp.where(skey <= t_s, jnp.float32(0.8), jnp.float32(0.2))
    out_ref[...] = out_ref[...] + jnp.sum(a_ref[0, 0] * wgt)

    @pl.when(jnp.logical_and(b == B - 1, q == NCHUNK - 1))
    def _():
        out_ref[...] = out_ref[...] * jnp.float32(1.0 / N_TOTAL)


def _loss(ts, keys, a):
    return pl.pallas_call(
        _loss_body,
        grid=(B, NCHUNK),
        in_specs=[
            pl.BlockSpec(memory_space=pltpu.SMEM),
            pl.BlockSpec((1, 1, ROWS, W), lambda b, q: (b, q, 0, 0)),
            pl.BlockSpec((1, 1, ROWS, W), lambda b, q: (b, q, 0, 0)),
        ],
        out_specs=pl.BlockSpec((1, 1), lambda b, q: (0, 0)),
        out_shape=jax.ShapeDtypeStruct((1, 1), jnp.float32),
    )(ts, keys, a)


def kernel(y_true, y_pred):
    keys, a = _prep(y_true, y_pred)
    ts = _select()(keys)
    loss = _loss(ts, keys, a)
    return jnp.reshape(loss, ())
